# baseline (speedup 1.0000x reference)
import functools

import jax
import jax.numpy as jnp
from jax import lax
from jax.experimental import pallas as pl
from jax.experimental.pallas import tpu as pltpu
from jax.experimental.pallas import tpu_sc as plsc

N = 10000
E = 320000
F = 128
D = 128
G = 512
OBS = 64

NC = 2
NS = 16
NW = NC * NS
RND = 5
KR = 2000
EPT = RND * KR
RT = 632
N_PAD = RT * NS

BN = 2000
GRID = N // BN


def _relu(v):
    return jnp.maximum(v, 0.0)


def _prep_body(x_ref, w1p_ref, b1p_ref, w2p_ref, b2p_ref, w3p_ref, b3p_ref,
               w1q_ref, b1q_ref, w2q_ref, b2q_ref, sel_ref, e9_ref, w1nx_ref,
               xprep_ref, a16_ref, h2s_ref, xn1_ref):
    xb = x_ref[...]
    h1 = _relu(xb @ w1p_ref[...] + b1p_ref[...])
    h2 = _relu(h1 @ w2p_ref[...] + b2p_ref[...])
    xp = h2 @ w3p_ref[...] + b3p_ref[...]
    xprep_ref[...] = xp
    t1 = _relu(xp @ w1q_ref[...] + b1q_ref[...])
    t2 = _relu(t1 @ w2q_ref[...] + b2q_ref[...])
    a16_ref[...] = t2 @ sel_ref[...] + e9_ref[...]
    h2s_ref[...] = h2 @ sel_ref[...]
    xn1_ref[...] = xb @ w1nx_ref[...]


def _post_body(h2s_ref, xn1_ref, parts_ref, batch_ref,
               w3ps_ref, b3p_ref, wc_ref, b1a_ref, w2a_ref, b2a_ref,
               w3a_ref, b3a_ref,
               w1ne_ref, b1n_ref, w2n_ref, b2n_ref, sel_ref, e9_ref,
               node_ref, dag_ref):
    acc = parts_ref[0] + parts_ref[1]
    g1 = _relu(acc @ wc_ref[...] + b1a_ref[...])
    g2 = _relu(g1 @ w2a_ref[...] + b2a_ref[...])
    xp = h2s_ref[...] @ w3ps_ref[...] + b3p_ref[...]
    ne = xp + (g2 @ w3a_ref[...] + b3a_ref[...])
    node_ref[...] = ne
    m1 = _relu(xn1_ref[...] + ne @ w1ne_ref[...] + b1n_ref[...])
    m2 = _relu(m1 @ w2n_ref[...] + b2n_ref[...])
    m16 = m2 @ sel_ref[...] + e9_ref[...]
    onehot = (lax.broadcasted_iota(jnp.int32, (G, BN), 0)
              == batch_ref[0]).astype(jnp.float32)
    contrib = lax.dot_general(onehot, m16, (((1,), (0,)), ((), ())))

    @pl.when(pl.program_id(0) == 0)
    def _():
        dag_ref[...] = contrib

    @pl.when(pl.program_id(0) != 0)
    def _():
        dag_ref[...] += contrib


def _dag_body(dacc_ref, w3n_ref, w1d_ref, b1d_ref, w2d_ref, b2d_ref,
              sel_ref, e9_ref, w3d_ref, lo_ref, hi_ref, dage_ref, z_ref):
    dage = dacc_ref[...] @ w3n_ref[...]
    dage_ref[...] = dage
    d1 = _relu(dage @ w1d_ref[...] + b1d_ref[...])
    d2 = _relu(d1 @ w2d_ref[...] + b2d_ref[...])
    d16 = d2 @ sel_ref[...] + e9_ref[...]
    gio = lax.broadcasted_iota(jnp.int32, (OBS, G), 1)
    mask = ((gio >= lo_ref[...]) & (gio < hi_ref[...])).astype(jnp.float32)
    z_ref[...] = (mask @ d16) @ w3d_ref[...]


def _sc_edge_scatter(table, edge_index):
    mesh = plsc.VectorSubcoreMesh(core_axis_name="c", subcore_axis_name="s")

    @functools.partial(
        pl.kernel,
        out_type=jax.ShapeDtypeStruct((NC, N_PAD, 16), jnp.float32),
        mesh=mesh,
        scratch_types=[
            pltpu.VMEM((EPT,), jnp.int32),
            pltpu.VMEM((RND, KR), jnp.int32),
            pltpu.VMEM((2, KR, 16), jnp.float32),
            pltpu.VMEM((RT, 16), jnp.float32),
            pltpu.VMEM_SHARED((N_PAD, 16), jnp.float32),
            pltpu.SemaphoreType.DMA,
            pltpu.SemaphoreType.DMA,
            pltpu.SemaphoreType.DMA,
        ],
        compiler_params=pltpu.CompilerParams(use_tc_tiling_on_sc=False),
    )
    def body(table_hbm, ei_hbm, out_hbm,
             src_v, dst_v, rows_v, zer_v, acc_sh, gsem, ssem0, ssem1):
        c = lax.axis_index("c")
        s = lax.axis_index("s")
        wid = s * NC + c
        base = wid * EPT

        def zrow(i, carry):
            zer_v[i, :] = jnp.zeros((16,), jnp.float32)
            return carry

        lax.fori_loop(0, RT, zrow, 0)
        pltpu.sync_copy(zer_v, acc_sh.at[pl.ds(s * RT, RT)])

        pltpu.sync_copy(ei_hbm.at[0, pl.ds(base, EPT)], src_v)
        for r in range(RND):
            pltpu.sync_copy(ei_hbm.at[1, pl.ds(base + r * KR, KR)],
                            dst_v.at[r])
        plsc.subcore_barrier()

        ssems = (ssem0, ssem1)
        scats = [None, None]
        for r in range(RND):
            slot = r % 2
            if scats[slot] is not None:
                scats[slot].wait()
                scats[slot] = None
            pltpu.async_copy(
                table_hbm.at[src_v.at[pl.ds(r * KR, KR)]],
                rows_v.at[slot], gsem).wait()
            scats[slot] = pltpu.async_copy(
                rows_v.at[slot], acc_sh.at[dst_v.at[r]], ssems[slot],
                add=True)
        scats[0].wait()
        scats[1].wait()
        plsc.subcore_barrier()

        pltpu.sync_copy(acc_sh.at[pl.ds(s * RT, RT)],
                        out_hbm.at[c, pl.ds(s * RT, RT)])

    return body(table, edge_index)


def _full(a):
    return pl.BlockSpec(a.shape, lambda i: (0,) * a.ndim)


def kernel(x, edge_index, batch, obs_indptr, prep, proc, agg, node, dag):
    w1p, b1p, w2p, b2p, w3p, b3p = prep
    w1q, b1q, w2q, b2q, w3q, b3q = proc
    w1a, b1a, w2a, b2a, w3a, b3a = agg
    w1n, b1n, w2n, b2n, w3n, b3n = node
    w1d, b1d, w2d, b2d, w3d, b3d = dag
    f32 = jnp.float32

    sel = jnp.concatenate([jnp.eye(8, dtype=f32), jnp.zeros((8, 8), f32)], 1)
    e9 = jnp.zeros((1, 16), f32).at[0, 8].set(1.0)
    zpad = jnp.zeros((7, D), f32)
    w3q_ext = jnp.concatenate([w3q, b3q[None, :], zpad], 0)
    wc = w3q_ext @ w1a
    w3n_ext = jnp.concatenate([w3n, b3n[None, :], zpad], 0)
    w3d_ext = jnp.concatenate([w3d, b3d[None, :], zpad], 0)
    r = lambda b: b.reshape(1, -1)

    a_ins = (x, w1p, r(b1p), w2p, r(b2p), w3p, r(b3p),
             w1q, r(b1q), w2q, r(b2q), sel, e9, w1n[:F])
    xprep, a16, h2s, xn1 = pl.pallas_call(
        _prep_body,
        grid=(GRID,),
        in_specs=[pl.BlockSpec((BN, F), lambda i: (i, 0))]
        + [_full(a) for a in a_ins[1:]],
        out_specs=[pl.BlockSpec((BN, D), lambda i: (i, 0)),
                   pl.BlockSpec((BN, 16), lambda i: (i, 0)),
                   pl.BlockSpec((BN, 16), lambda i: (i, 0)),
                   pl.BlockSpec((BN, 16), lambda i: (i, 0))],
        out_shape=[jax.ShapeDtypeStruct((N, D), f32),
                   jax.ShapeDtypeStruct((N, 16), f32),
                   jax.ShapeDtypeStruct((N, 16), f32),
                   jax.ShapeDtypeStruct((N, 16), f32)],
    )(*a_ins)

    parts = _sc_edge_scatter(a16, edge_index)

    w3ps = jnp.concatenate([w3p, jnp.zeros((8, D), f32)], 0)
    c_ins = (h2s, xn1, parts, batch.reshape(GRID, 1, BN),
             w3ps, r(b3p), wc, r(b1a), w2a, r(b2a), w3a, r(b3a),
             w1n[F:], r(b1n), w2n, r(b2n), sel, e9)
    node_emb, dagacc = pl.pallas_call(
        _post_body,
        grid=(GRID,),
        in_specs=[pl.BlockSpec((BN, 16), lambda i: (i, 0)),
                  pl.BlockSpec((BN, 16), lambda i: (i, 0)),
                  pl.BlockSpec((NC, BN, 16), lambda i: (0, i, 0)),
                  pl.BlockSpec((1, 1, BN), lambda i: (i, 0, 0))]
        + [_full(a) for a in c_ins[4:]],
        out_specs=[pl.BlockSpec((BN, D), lambda i: (i, 0)),
                   pl.BlockSpec((G, 16), lambda i: (0, 0))],
        out_shape=[jax.ShapeDtypeStruct((N, D), f32),
                   jax.ShapeDtypeStruct((G, 16), f32)],
    )(*c_ins)

    lo = obs_indptr[:-1].reshape(OBS, 1)
    hi = obs_indptr[1:].reshape(OBS, 1)
    dag_emb, z = pl.pallas_call(
        _dag_body,
        out_shape=[jax.ShapeDtypeStruct((G, D), f32),
                   jax.ShapeDtypeStruct((OBS, D), f32)],
    )(dagacc, w3n_ext, w1d, r(b1d), w2d, r(b2d), sel, e9, w3d_ext, lo, hi)

    return node_emb, dag_emb, z


# speedup vs baseline: 1.0703x; 1.0703x over previous
import functools

import jax
import jax.numpy as jnp
from jax import lax
from jax.experimental import pallas as pl
from jax.experimental.pallas import tpu as pltpu
from jax.experimental.pallas import tpu_sc as plsc

N = 10000
E = 320000
F = 128
D = 128
G = 512
OBS = 64

NC = 2
NS = 16
NW = NC * NS
RND = 5
KR = 2000
EPT = RND * KR
RT = 632
N_PAD = RT * NS

BN = 2000
GRID = N // BN


def _relu(v):
    return jnp.maximum(v, 0.0)


def _prep_body(x_ref, w1p_ref, b1p_ref, w2p_ref, b2p_ref, w3p_ref, b3p_ref,
               w1q_ref, b1q_ref, w2q_ref, b2q_ref, sel_ref, e9_ref,
               xprep_ref, a16_ref):
    xb = x_ref[...]
    h1 = _relu(xb @ w1p_ref[...] + b1p_ref[...])
    h2 = _relu(h1 @ w2p_ref[...] + b2p_ref[...])
    xp = h2 @ w3p_ref[...] + b3p_ref[...]
    xprep_ref[...] = xp
    t1 = _relu(xp @ w1q_ref[...] + b1q_ref[...])
    t2 = _relu(t1 @ w2q_ref[...] + b2q_ref[...])
    a16_ref[...] = t2 @ sel_ref[...] + e9_ref[...]


def _post_body(x_ref, xprep_ref, parts_ref, batch_ref,
               wc_ref, b1a_ref, w2a_ref, b2a_ref, w3a_ref, b3a_ref,
               w1nx_ref, w1ne_ref, b1n_ref, w2n_ref, b2n_ref, sel_ref, e9_ref,
               node_ref, dag_ref):
    acc = parts_ref[0] + parts_ref[1]
    g1 = _relu(acc @ wc_ref[...] + b1a_ref[...])
    g2 = _relu(g1 @ w2a_ref[...] + b2a_ref[...])
    ne = xprep_ref[...] + (g2 @ w3a_ref[...] + b3a_ref[...])
    node_ref[...] = ne
    m1 = _relu(x_ref[...] @ w1nx_ref[...] + ne @ w1ne_ref[...] + b1n_ref[...])
    m2 = _relu(m1 @ w2n_ref[...] + b2n_ref[...])
    m16 = m2 @ sel_ref[...] + e9_ref[...]
    onehot = (lax.broadcasted_iota(jnp.int32, (G, BN), 0)
              == batch_ref[0]).astype(jnp.float32)
    contrib = lax.dot_general(onehot, m16, (((1,), (0,)), ((), ())))

    @pl.when(pl.program_id(0) == 0)
    def _():
        dag_ref[...] = contrib

    @pl.when(pl.program_id(0) != 0)
    def _():
        dag_ref[...] += contrib


def _dag_body(dacc_ref, w3n_ref, w1d_ref, b1d_ref, w2d_ref, b2d_ref,
              sel_ref, e9_ref, w3d_ref, lo_ref, hi_ref, dage_ref, z_ref):
    dage = dacc_ref[...] @ w3n_ref[...]
    dage_ref[...] = dage
    d1 = _relu(dage @ w1d_ref[...] + b1d_ref[...])
    d2 = _relu(d1 @ w2d_ref[...] + b2d_ref[...])
    d16 = d2 @ sel_ref[...] + e9_ref[...]
    gio = lax.broadcasted_iota(jnp.int32, (OBS, G), 1)
    mask = ((gio >= lo_ref[...]) & (gio < hi_ref[...])).astype(jnp.float32)
    z_ref[...] = (mask @ d16) @ w3d_ref[...]


def _sc_edge_scatter(table, edge_index):
    mesh = plsc.VectorSubcoreMesh(core_axis_name="c", subcore_axis_name="s")

    @functools.partial(
        pl.kernel,
        out_type=jax.ShapeDtypeStruct((NC, N_PAD, 16), jnp.float32),
        mesh=mesh,
        scratch_types=[
            pltpu.VMEM((EPT,), jnp.int32),
            pltpu.VMEM((RND, KR), jnp.int32),
            pltpu.VMEM((2, KR, 16), jnp.float32),
            pltpu.VMEM((RT, 16), jnp.float32),
            pltpu.VMEM_SHARED((N_PAD, 16), jnp.float32),
            pltpu.SemaphoreType.DMA,
            pltpu.SemaphoreType.DMA,
            pltpu.SemaphoreType.DMA,
            pltpu.SemaphoreType.DMA,
        ],
        compiler_params=pltpu.CompilerParams(use_tc_tiling_on_sc=False),
    )
    def body(table_hbm, ei_hbm, out_hbm,
             src_v, dst_v, rows_v, zer_v, acc_sh, gsem, gsem2, ssem0, ssem1):
        c = lax.axis_index("c")
        s = lax.axis_index("s")
        wid = s * NC + c
        base = wid * EPT

        stg = [pltpu.async_copy(ei_hbm.at[0, pl.ds(base, EPT)], src_v, gsem)]
        for r in range(RND):
            stg.append(pltpu.async_copy(
                ei_hbm.at[1, pl.ds(base + r * KR, KR)], dst_v.at[r], gsem))

        def zrow(i, carry):
            zer_v[i, :] = jnp.zeros((16,), jnp.float32)
            return carry

        lax.fori_loop(0, RT, zrow, 0)
        pltpu.sync_copy(zer_v, acc_sh.at[pl.ds(s * RT, RT)])
        for d in stg:
            d.wait()

        ssems = (ssem0, ssem1)
        gsems = (gsem, gsem2)
        scats = [None, None]
        gets = [None, None]
        gets[0] = pltpu.async_copy(
            table_hbm.at[src_v.at[pl.ds(0, KR)]], rows_v.at[0], gsems[0])
        for r in range(RND):
            slot = r % 2
            nslot = (r + 1) % 2
            if r + 1 < RND:
                if scats[nslot] is not None:
                    scats[nslot].wait()
                    scats[nslot] = None
                gets[nslot] = pltpu.async_copy(
                    table_hbm.at[src_v.at[pl.ds((r + 1) * KR, KR)]],
                    rows_v.at[nslot], gsems[nslot])
            gets[slot].wait()
            if r == 0:
                plsc.subcore_barrier()
            scats[slot] = pltpu.async_copy(
                rows_v.at[slot], acc_sh.at[dst_v.at[r]], ssems[slot],
                add=True)
        scats[0].wait()
        scats[1].wait()
        plsc.subcore_barrier()

        pltpu.sync_copy(acc_sh.at[pl.ds(s * RT, RT)],
                        out_hbm.at[c, pl.ds(s * RT, RT)])

    return body(table, edge_index)


def _full(a):
    return pl.BlockSpec(a.shape, lambda i: (0,) * a.ndim)


def kernel(x, edge_index, batch, obs_indptr, prep, proc, agg, node, dag):
    w1p, b1p, w2p, b2p, w3p, b3p = prep
    w1q, b1q, w2q, b2q, w3q, b3q = proc
    w1a, b1a, w2a, b2a, w3a, b3a = agg
    w1n, b1n, w2n, b2n, w3n, b3n = node
    w1d, b1d, w2d, b2d, w3d, b3d = dag
    f32 = jnp.float32

    sel = jnp.concatenate([jnp.eye(8, dtype=f32), jnp.zeros((8, 8), f32)], 1)
    e9 = jnp.zeros((1, 16), f32).at[0, 8].set(1.0)
    zpad = jnp.zeros((7, D), f32)
    w3q_ext = jnp.concatenate([w3q, b3q[None, :], zpad], 0)
    wc = w3q_ext @ w1a
    w3n_ext = jnp.concatenate([w3n, b3n[None, :], zpad], 0)
    w3d_ext = jnp.concatenate([w3d, b3d[None, :], zpad], 0)
    r = lambda b: b.reshape(1, -1)

    a_ins = (x, w1p, r(b1p), w2p, r(b2p), w3p, r(b3p),
             w1q, r(b1q), w2q, r(b2q), sel, e9)
    xprep, a16 = pl.pallas_call(
        _prep_body,
        grid=(GRID,),
        in_specs=[pl.BlockSpec((BN, F), lambda i: (i, 0))]
        + [_full(a) for a in a_ins[1:]],
        out_specs=[pl.BlockSpec((BN, D), lambda i: (i, 0)),
                   pl.BlockSpec((BN, 16), lambda i: (i, 0))],
        out_shape=[jax.ShapeDtypeStruct((N, D), f32),
                   jax.ShapeDtypeStruct((N, 16), f32)],
    )(*a_ins)

    parts = _sc_edge_scatter(a16, edge_index)

    c_ins = (x, xprep, parts, batch.reshape(GRID, 1, BN),
             wc, r(b1a), w2a, r(b2a), w3a, r(b3a),
             w1n[:F], w1n[F:], r(b1n), w2n, r(b2n), sel, e9)
    node_emb, dagacc = pl.pallas_call(
        _post_body,
        grid=(GRID,),
        in_specs=[pl.BlockSpec((BN, F), lambda i: (i, 0)),
                  pl.BlockSpec((BN, D), lambda i: (i, 0)),
                  pl.BlockSpec((NC, BN, 16), lambda i: (0, i, 0)),
                  pl.BlockSpec((1, 1, BN), lambda i: (i, 0, 0))]
        + [_full(a) for a in c_ins[4:]],
        out_specs=[pl.BlockSpec((BN, D), lambda i: (i, 0)),
                   pl.BlockSpec((G, 16), lambda i: (0, 0))],
        out_shape=[jax.ShapeDtypeStruct((N, D), f32),
                   jax.ShapeDtypeStruct((G, 16), f32)],
    )(*c_ins)

    lo = obs_indptr[:-1].reshape(OBS, 1)
    hi = obs_indptr[1:].reshape(OBS, 1)
    dag_emb, z = pl.pallas_call(
        _dag_body,
        out_shape=[jax.ShapeDtypeStruct((G, D), f32),
                   jax.ShapeDtypeStruct((OBS, D), f32)],
    )(dagacc, w3n_ext, w1d, r(b1d), w2d, r(b2d), sel, e9, w3d_ext, lo, hi)

    return node_emb, dag_emb, z


# speedup vs baseline: 1.0809x; 1.0099x over previous
import functools

import jax
import jax.numpy as jnp
from jax import lax
from jax.experimental import pallas as pl
from jax.experimental.pallas import tpu as pltpu
from jax.experimental.pallas import tpu_sc as plsc

N = 10000
E = 320000
F = 128
D = 128
G = 512
OBS = 64

NC = 2
NS = 16
NW = NC * NS
RND = 5
KR = 2000
EPT = RND * KR
RT = 632
N_PAD = RT * NS

BN = 2000
GRID = N // BN


def _relu(v):
    return jnp.maximum(v, 0.0)


def _prep_body(x_ref, w1p_ref, b1p_ref, w2p_ref, b2p_ref, w3p_ref, b3p_ref,
               w1q_ref, b1q_ref, w2q_ref, b2q_ref, sel_ref, e9_ref,
               xprep_ref, a16_ref):
    xb = x_ref[...]
    h1 = _relu(xb @ w1p_ref[...] + b1p_ref[...])
    h2 = _relu(h1 @ w2p_ref[...] + b2p_ref[...])
    xp = h2 @ w3p_ref[...] + b3p_ref[...]
    xprep_ref[...] = xp
    t1 = _relu(xp @ w1q_ref[...] + b1q_ref[...])
    t2 = _relu(t1 @ w2q_ref[...] + b2q_ref[...])
    a16_ref[...] = t2 @ sel_ref[...] + e9_ref[...]


def _post_body(x_ref, xprep_ref, parts_ref, batch_ref,
               wc_ref, b1a_ref, w2a_ref, b2a_ref, w3a_ref, b3a_ref,
               w1nx_ref, w1ne_ref, b1n_ref, w2n_ref, b2n_ref, sel_ref, e9_ref,
               w3n_ref, w1d_ref, b1d_ref, w2d_ref, b2d_ref, w3d_ref,
               lo_ref, hi_ref,
               node_ref, dage_ref, z_ref, dag_ref):
    acc = parts_ref[0] + parts_ref[1]
    g1 = _relu(acc @ wc_ref[...] + b1a_ref[...])
    g2 = _relu(g1 @ w2a_ref[...] + b2a_ref[...])
    ne = xprep_ref[...] + (g2 @ w3a_ref[...] + b3a_ref[...])
    node_ref[...] = ne
    m1 = _relu(x_ref[...] @ w1nx_ref[...] + ne @ w1ne_ref[...] + b1n_ref[...])
    m2 = _relu(m1 @ w2n_ref[...] + b2n_ref[...])
    m16 = m2 @ sel_ref[...] + e9_ref[...]
    onehot = (lax.broadcasted_iota(jnp.int32, (G, BN), 0)
              == batch_ref[0]).astype(jnp.float32)
    contrib = lax.dot_general(onehot, m16, (((1,), (0,)), ((), ())))

    @pl.when(pl.program_id(0) == 0)
    def _():
        dag_ref[...] = contrib

    @pl.when(pl.program_id(0) != 0)
    def _():
        dag_ref[...] += contrib

    @pl.when(pl.program_id(0) == GRID - 1)
    def _():
        dage = dag_ref[...] @ w3n_ref[...]
        dage_ref[...] = dage
        d1 = _relu(dage @ w1d_ref[...] + b1d_ref[...])
        d2 = _relu(d1 @ w2d_ref[...] + b2d_ref[...])
        d16 = d2 @ sel_ref[...] + e9_ref[...]
        gio = lax.broadcasted_iota(jnp.int32, (OBS, G), 1)
        mask = ((gio >= lo_ref[...]) & (gio < hi_ref[...])).astype(jnp.float32)
        z_ref[...] = (mask @ d16) @ w3d_ref[...]


def _dag_body(dacc_ref, w3n_ref, w1d_ref, b1d_ref, w2d_ref, b2d_ref,
              sel_ref, e9_ref, w3d_ref, lo_ref, hi_ref, dage_ref, z_ref):
    dage = dacc_ref[...] @ w3n_ref[...]
    dage_ref[...] = dage
    d1 = _relu(dage @ w1d_ref[...] + b1d_ref[...])
    d2 = _relu(d1 @ w2d_ref[...] + b2d_ref[...])
    d16 = d2 @ sel_ref[...] + e9_ref[...]
    gio = lax.broadcasted_iota(jnp.int32, (OBS, G), 1)
    mask = ((gio >= lo_ref[...]) & (gio < hi_ref[...])).astype(jnp.float32)
    z_ref[...] = (mask @ d16) @ w3d_ref[...]


def _sc_edge_scatter(table, edge_index):
    mesh = plsc.VectorSubcoreMesh(core_axis_name="c", subcore_axis_name="s")

    @functools.partial(
        pl.kernel,
        out_type=jax.ShapeDtypeStruct((NC, N_PAD, 16), jnp.float32),
        mesh=mesh,
        scratch_types=[
            pltpu.VMEM((EPT,), jnp.int32),
            pltpu.VMEM((RND, KR), jnp.int32),
            pltpu.VMEM((2, KR, 16), jnp.float32),
            pltpu.VMEM((RT, 16), jnp.float32),
            pltpu.VMEM_SHARED((N_PAD, 16), jnp.float32),
            pltpu.SemaphoreType.DMA,
            pltpu.SemaphoreType.DMA,
            pltpu.SemaphoreType.DMA,
            pltpu.SemaphoreType.DMA,
        ],
        compiler_params=pltpu.CompilerParams(use_tc_tiling_on_sc=False),
    )
    def body(table_hbm, ei_hbm, out_hbm,
             src_v, dst_v, rows_v, zer_v, acc_sh, gsem, gsem2, ssem0, ssem1):
        c = lax.axis_index("c")
        s = lax.axis_index("s")
        wid = s * NC + c
        base = wid * EPT

        stg = [pltpu.async_copy(ei_hbm.at[0, pl.ds(base, EPT)], src_v, gsem)]
        for r in range(RND):
            stg.append(pltpu.async_copy(
                ei_hbm.at[1, pl.ds(base + r * KR, KR)], dst_v.at[r], gsem))

        def zrow(i, carry):
            zer_v[i, :] = jnp.zeros((16,), jnp.float32)
            return carry

        lax.fori_loop(0, RT, zrow, 0)
        pltpu.sync_copy(zer_v, acc_sh.at[pl.ds(s * RT, RT)])
        for d in stg:
            d.wait()

        ssems = (ssem0, ssem1)
        gsems = (gsem, gsem2)
        scats = [None, None]
        gets = [None, None]
        gets[0] = pltpu.async_copy(
            table_hbm.at[src_v.at[pl.ds(0, KR)]], rows_v.at[0], gsems[0])
        for r in range(RND):
            slot = r % 2
            nslot = (r + 1) % 2
            if r + 1 < RND:
                if scats[nslot] is not None:
                    scats[nslot].wait()
                    scats[nslot] = None
                gets[nslot] = pltpu.async_copy(
                    table_hbm.at[src_v.at[pl.ds((r + 1) * KR, KR)]],
                    rows_v.at[nslot], gsems[nslot])
            gets[slot].wait()
            if r == 0:
                plsc.subcore_barrier()
            scats[slot] = pltpu.async_copy(
                rows_v.at[slot], acc_sh.at[dst_v.at[r]], ssems[slot],
                add=True)
        scats[0].wait()
        scats[1].wait()
        plsc.subcore_barrier()

        pltpu.sync_copy(acc_sh.at[pl.ds(s * RT, RT)],
                        out_hbm.at[c, pl.ds(s * RT, RT)])

    return body(table, edge_index)


def _full(a):
    return pl.BlockSpec(a.shape, lambda i: (0,) * a.ndim)


def kernel(x, edge_index, batch, obs_indptr, prep, proc, agg, node, dag):
    w1p, b1p, w2p, b2p, w3p, b3p = prep
    w1q, b1q, w2q, b2q, w3q, b3q = proc
    w1a, b1a, w2a, b2a, w3a, b3a = agg
    w1n, b1n, w2n, b2n, w3n, b3n = node
    w1d, b1d, w2d, b2d, w3d, b3d = dag
    f32 = jnp.float32

    sel = jnp.concatenate([jnp.eye(8, dtype=f32), jnp.zeros((8, 8), f32)], 1)
    e9 = jnp.zeros((1, 16), f32).at[0, 8].set(1.0)
    zpad = jnp.zeros((7, D), f32)
    w3q_ext = jnp.concatenate([w3q, b3q[None, :], zpad], 0)
    wc = w3q_ext @ w1a
    w3n_ext = jnp.concatenate([w3n, b3n[None, :], zpad], 0)
    w3d_ext = jnp.concatenate([w3d, b3d[None, :], zpad], 0)
    r = lambda b: b.reshape(1, -1)

    a_ins = (x, w1p, r(b1p), w2p, r(b2p), w3p, r(b3p),
             w1q, r(b1q), w2q, r(b2q), sel, e9)
    xprep, a16 = pl.pallas_call(
        _prep_body,
        grid=(GRID,),
        in_specs=[pl.BlockSpec((BN, F), lambda i: (i, 0))]
        + [_full(a) for a in a_ins[1:]],
        out_specs=[pl.BlockSpec((BN, D), lambda i: (i, 0)),
                   pl.BlockSpec((BN, 16), lambda i: (i, 0))],
        out_shape=[jax.ShapeDtypeStruct((N, D), f32),
                   jax.ShapeDtypeStruct((N, 16), f32)],
    )(*a_ins)

    parts = _sc_edge_scatter(a16, edge_index)

    lo = obs_indptr[:-1].reshape(OBS, 1)
    hi = obs_indptr[1:].reshape(OBS, 1)
    c_ins = (x, xprep, parts, batch.reshape(GRID, 1, BN),
             wc, r(b1a), w2a, r(b2a), w3a, r(b3a),
             w1n[:F], w1n[F:], r(b1n), w2n, r(b2n), sel, e9,
             w3n_ext, w1d, r(b1d), w2d, r(b2d), w3d_ext, lo, hi)
    node_emb, dag_emb, z, _ = pl.pallas_call(
        _post_body,
        grid=(GRID,),
        in_specs=[pl.BlockSpec((BN, F), lambda i: (i, 0)),
                  pl.BlockSpec((BN, D), lambda i: (i, 0)),
                  pl.BlockSpec((NC, BN, 16), lambda i: (0, i, 0)),
                  pl.BlockSpec((1, 1, BN), lambda i: (i, 0, 0))]
        + [_full(a) for a in c_ins[4:]],
        out_specs=[pl.BlockSpec((BN, D), lambda i: (i, 0)),
                   pl.BlockSpec((G, D), lambda i: (0, 0)),
                   pl.BlockSpec((OBS, D), lambda i: (0, 0)),
                   pl.BlockSpec((G, 16), lambda i: (0, 0))],
        out_shape=[jax.ShapeDtypeStruct((N, D), f32),
                   jax.ShapeDtypeStruct((G, D), f32),
                   jax.ShapeDtypeStruct((OBS, D), f32),
                   jax.ShapeDtypeStruct((G, 16), f32)],
    )(*c_ins)

    return node_emb, dag_emb, z
